# Initial kernel scaffold; baseline (speedup 1.0000x reference)
import jax
import jax.numpy as jnp
from jax.experimental import pallas as pl


def kernel(x, edge_index, W1, b1, W2, b2, Wl, bl):
    raise NotImplementedError("write your pallas kernel here")


# speedup vs baseline: 16.2214x; 16.2214x over previous
import functools

import jax
import jax.numpy as jnp
from jax import lax
from jax.experimental import pallas as pl
from jax.experimental.pallas import tpu as pltpu
from jax.experimental.pallas import tpu_sc as plsc

N_NODES = 10000
D = 128
N_EDGES = 320000

NC = 2
NS = 16
NW = NC * NS

NP = 10240
ROWS_PER_TILE = NP // NS
EDGES_PER_TILE = N_EDGES // NW

CHUNK = 128
N_CHUNK = EDGES_PER_TILE // CHUNK
REM = EDGES_PER_TILE - N_CHUNK * CHUNK

HCHUNK = 80
N_HCHUNK = EDGES_PER_TILE // HCHUNK

def _zero_vmem_2d(ref, rows):
    z16 = jnp.zeros((16,), jnp.float32)

    def body(i, carry):
        for j in range(8):
            ref[i, pl.ds(j * 16, 16)] = z16
        return carry

    lax.fori_loop(0, rows, body, 0)


@functools.cache
def _make_sc_hist():
    mesh = plsc.VectorSubcoreMesh(core_axis_name="c", subcore_axis_name="s")
    return pl.kernel(
        _sc_hist,
        out_type=jax.ShapeDtypeStruct((NC, NP), jnp.float32),
        mesh=mesh,
        scratch_types=[
            pltpu.VMEM((HCHUNK,), jnp.int32),
            pltpu.VMEM((HCHUNK,), jnp.float32),
            pltpu.VMEM((ROWS_PER_TILE,), jnp.float32),
            pltpu.VMEM_SHARED((NP,), jnp.float32),
        ],
    )


def _sc_hist(dst_hbm, out_hbm, idx_v, ones_v, zero_v, hist_sh):
    c = lax.axis_index("c")
    s = lax.axis_index("s")
    wid = c * NS + s

    one16 = jnp.ones((16,), jnp.float32)
    z16 = jnp.zeros((16,), jnp.float32)
    for j in range(HCHUNK // 16):
        ones_v[pl.ds(j * 16, 16)] = one16

    def zb(i, carry):
        zero_v[pl.ds(i * 16, 16)] = z16
        return carry

    lax.fori_loop(0, ROWS_PER_TILE // 16, zb, 0)
    pltpu.sync_copy(zero_v, hist_sh.at[pl.ds(s * ROWS_PER_TILE, ROWS_PER_TILE)])
    plsc.subcore_barrier()

    base = wid * EDGES_PER_TILE

    def eb(j, carry):
        pltpu.sync_copy(dst_hbm.at[pl.ds(base + j * HCHUNK, HCHUNK)], idx_v)
        pltpu.sync_copy(ones_v, hist_sh.at[idx_v], add=True)
        return carry

    lax.fori_loop(0, N_HCHUNK, eb, 0)
    plsc.subcore_barrier()
    pltpu.sync_copy(
        hist_sh.at[pl.ds(s * ROWS_PER_TILE, ROWS_PER_TILE)],
        out_hbm.at[c, pl.ds(s * ROWS_PER_TILE, ROWS_PER_TILE)],
    )


@functools.cache
def _make_sc_agg():
    mesh = plsc.VectorSubcoreMesh(core_axis_name="c", subcore_axis_name="s")
    return pl.kernel(
        _sc_agg,
        out_type=jax.ShapeDtypeStruct((NC, NP, D), jnp.float32),
        mesh=mesh,
        scratch_types=[
            pltpu.VMEM((CHUNK,), jnp.int32),
            pltpu.VMEM((CHUNK,), jnp.int32),
            pltpu.VMEM((CHUNK, D), jnp.float32),
            pltpu.VMEM((REM,), jnp.int32),
            pltpu.VMEM((REM,), jnp.int32),
            pltpu.VMEM((REM, D), jnp.float32),
            pltpu.VMEM((64, D), jnp.float32),
            pltpu.VMEM_SHARED((NP, D), jnp.float32),
            pltpu.SemaphoreType.DMA,
        ],
    )


def _sc_agg(y_hbm, src_hbm, dst_hbm, out_hbm,
            src_v, dst_v, rows_v, src16_v, dst16_v, rows16_v, zrow_v,
            acc_sh, sem):
    c = lax.axis_index("c")
    s = lax.axis_index("s")
    wid = c * NS + s

    _zero_vmem_2d(zrow_v, 64)

    def zc(i, carry):
        pltpu.sync_copy(zrow_v, acc_sh.at[pl.ds(s * ROWS_PER_TILE + i * 64, 64)])
        return carry

    lax.fori_loop(0, ROWS_PER_TILE // 64, zc, 0)
    plsc.subcore_barrier()

    base = wid * EDGES_PER_TILE

    def eb(j, carry):
        off = base + j * CHUNK
        pltpu.sync_copy(src_hbm.at[pl.ds(off, CHUNK)], src_v)
        pltpu.sync_copy(dst_hbm.at[pl.ds(off, CHUNK)], dst_v)
        pltpu.async_copy(y_hbm.at[src_v], rows_v, sem).wait()
        pltpu.sync_copy(rows_v, acc_sh.at[dst_v], add=True)
        return carry

    lax.fori_loop(0, N_CHUNK, eb, 0)

    off = base + N_CHUNK * CHUNK
    pltpu.sync_copy(src_hbm.at[pl.ds(off, REM)], src16_v)
    pltpu.sync_copy(dst_hbm.at[pl.ds(off, REM)], dst16_v)
    pltpu.async_copy(y_hbm.at[src16_v], rows16_v, sem).wait()
    pltpu.sync_copy(rows16_v, acc_sh.at[dst16_v], add=True)

    plsc.subcore_barrier()
    pltpu.sync_copy(
        acc_sh.at[pl.ds(s * ROWS_PER_TILE, ROWS_PER_TILE)],
        out_hbm.at[c, pl.ds(s * ROWS_PER_TILE, ROWS_PER_TILE)],
    )


def _disb_from_hist(hist_ref):
    p = hist_ref[...]
    deg = p[0:1, :] + p[1:2, :] + 1.0
    dis = lax.rsqrt(deg)
    ones = jnp.ones((1, D), jnp.float32)
    return lax.dot_general(dis, ones, (((0,), (0,)), ((), ())),
                           preferred_element_type=jnp.float32)


def _tc_prep(hist, x, W1):
    def body(hist_ref, x_ref, w_ref, y_ref, disb_ref):
        disb = _disb_from_hist(hist_ref)
        disb_ref[...] = disb
        xw = jnp.dot(x_ref[...], w_ref[...], preferred_element_type=jnp.float32)
        y_ref[...] = xw * disb[:N_NODES]

    return pl.pallas_call(
        body,
        out_shape=(
            jax.ShapeDtypeStruct((N_NODES, D), jnp.float32),
            jax.ShapeDtypeStruct((NP, D), jnp.float32),
        ),
    )(hist, x, W1)


def _tc_mid(disb, agg, y, b, W2):
    def body(disb_ref, agg_ref, y_ref, b_ref, w_ref, y2_ref):
        disb = disb_ref[...][:N_NODES]
        a = agg_ref[0, :N_NODES, :] + agg_ref[1, :N_NODES, :] + y_ref[...]
        h = jnp.maximum(disb * a + b_ref[...], 0.0)
        hw = jnp.dot(h, w_ref[...], preferred_element_type=jnp.float32)
        y2_ref[...] = hw * disb

    return pl.pallas_call(
        body,
        out_shape=jax.ShapeDtypeStruct((N_NODES, D), jnp.float32),
    )(disb, agg, y, b, W2)


def _tc_out(disb, agg, y, b, Wl, bl):
    def body(disb_ref, agg_ref, y_ref, b_ref, wl_ref, bl_ref, out_ref):
        disb = disb_ref[...][:N_NODES]
        a = agg_ref[0, :N_NODES, :] + agg_ref[1, :N_NODES, :] + y_ref[...]
        h = disb * a + b_ref[...]
        out_ref[...] = jnp.dot(h, wl_ref[...],
                               preferred_element_type=jnp.float32) + bl_ref[...]

    return pl.pallas_call(
        body,
        out_shape=jax.ShapeDtypeStruct((N_NODES, 1), jnp.float32),
    )(disb, agg, y, b, Wl, bl)


def kernel(x, edge_index, W1, b1, W2, b2, Wl, bl):
    src = edge_index[0]
    dst = edge_index[1]

    sc_hist = _make_sc_hist()
    sc_agg = _make_sc_agg()
    hist = sc_hist(dst)
    y1, disb = _tc_prep(hist, x, W1)
    agg1 = sc_agg(y1, src, dst)
    y2 = _tc_mid(disb, agg1, y1, b1.reshape(1, D), W2)
    agg2 = sc_agg(y2, src, dst)
    out = _tc_out(disb, agg2, y2, b2.reshape(1, D),
                  Wl, bl.reshape(1, 1))
    return jnp.squeeze(out, axis=-1)
